# baseline (speedup 1.0000x reference)
import functools

import jax
import jax.numpy as jnp
from jax import lax
from jax.experimental import pallas as pl
from jax.experimental.pallas import tpu as pltpu
from jax.experimental.pallas import tpu_sc as plsc

_NUM_CORES = 2
_NUM_SUBCORES = 16
_NUM_WORKERS = _NUM_CORES * _NUM_SUBCORES

_CHUNK = 128
_NBUF = 5
_OUTW = 128


def _make_sc_gather(n_idx: int, emb: int):
    assert n_idx % (_NUM_WORKERS * _CHUNK * _NBUF) == 0
    per_w = n_idx // _NUM_WORKERS
    groups = per_w // (_CHUNK * _NBUF)

    mesh = plsc.VectorSubcoreMesh(core_axis_name="c", subcore_axis_name="s")

    @functools.partial(
        pl.kernel,
        mesh=mesh,
        out_type=jax.ShapeDtypeStruct((n_idx, _OUTW), jnp.float32),
        compiler_params=pltpu.CompilerParams(use_tc_tiling_on_sc=False),
        scratch_types=[
            pltpu.VMEM((per_w,), jnp.int32),
            [pltpu.VMEM((_CHUNK, emb), jnp.float32) for _ in range(_NBUF)],
            [pltpu.SemaphoreType.DMA for _ in range(_NBUF)],
        ],
    )
    def gather_kernel(table_hbm, idx_hbm, out_hbm, idx_v, rows, sems):
        wid = lax.axis_index("s") * _NUM_CORES + lax.axis_index("c")
        base = wid * per_w
        pltpu.sync_copy(idx_hbm.at[pl.ds(base, per_w)], idx_v)

        def group_body(g, carry):
            goff = g * (_CHUNK * _NBUF)
            copies = []
            for b in range(_NBUF):
                off = goff + b * _CHUNK
                copies.append(
                    pltpu.async_copy(
                        table_hbm.at[idx_v.at[pl.ds(off, _CHUNK)]],
                        rows[b],
                        sems[b],
                    )
                )
            for b in range(_NBUF):
                off = goff + b * _CHUNK
                copies[b].wait()
                pltpu.sync_copy(
                    rows[b],
                    out_hbm.at[pl.ds(base + off, _CHUNK), pl.ds(0, emb)],
                )
            return carry

        lax.fori_loop(0, groups, group_body, 0)

    return gather_kernel


def _rnn_step(emb_ref, wih_ref, whh_ref, b_ref, out_ref, embout_ref, h_ref):
    t = pl.program_id(0)

    @pl.when(t == 0)
    def _():
        h_ref[...] = jnp.zeros_like(h_ref)

    hid = out_ref.shape[2]
    x = emb_ref[0][:, :hid]
    embout_ref[0] = x
    h = h_ref[...]
    pre = (
        jnp.dot(x, wih_ref[...], preferred_element_type=jnp.float32)
        + jnp.dot(h, whh_ref[...], preferred_element_type=jnp.float32)
        + b_ref[...]
    )
    h_new = jnp.tanh(pre)
    h_ref[...] = h_new
    out_ref[0] = h_new


def _make_tc_rnn(seq: int, batch: int, emb: int, hid: int):
    return pl.pallas_call(
        _rnn_step,
        grid=(seq,),
        in_specs=[
            pl.BlockSpec((1, batch, _OUTW), lambda t: (t, 0, 0)),
            pl.BlockSpec((emb, hid), lambda t: (0, 0)),
            pl.BlockSpec((hid, hid), lambda t: (0, 0)),
            pl.BlockSpec((1, hid), lambda t: (0, 0)),
        ],
        out_specs=[
            pl.BlockSpec((1, batch, hid), lambda t: (t, 0, 0)),
            pl.BlockSpec((1, batch, emb), lambda t: (t, 0, 0)),
        ],
        out_shape=[
            jax.ShapeDtypeStruct((seq, batch, hid), jnp.float32),
            jax.ShapeDtypeStruct((seq, batch, emb), jnp.float32),
        ],
        scratch_shapes=[pltpu.VMEM((batch, hid), jnp.float32)],
    )


def kernel(input_seq, emb_table, W_ih, W_hh, b_ih, b_hh):
    seq, batch = input_seq.shape
    vocab, emb = emb_table.shape
    hid = W_hh.shape[0]

    idx_flat = input_seq.reshape(-1).astype(jnp.int32)
    gathered = _make_sc_gather(seq * batch, emb)(emb_table, idx_flat)
    emb_wide = gathered.reshape(seq, batch, _OUTW)

    bias = (b_ih + b_hh).reshape(1, hid)
    output_seq, embedded_seq = _make_tc_rnn(seq, batch, emb, hid)(
        emb_wide, W_ih.T, W_hh.T, bias
    )
    last_hidden = output_seq[seq - 1 : seq]
    return output_seq, last_hidden, embedded_seq


# speedup vs baseline: 1.1378x; 1.1378x over previous
import functools

import jax
import jax.numpy as jnp
from jax import lax
from jax.experimental import pallas as pl
from jax.experimental.pallas import tpu as pltpu
from jax.experimental.pallas import tpu_sc as plsc

_NUM_CORES = 2
_NUM_SUBCORES = 16
_NUM_WORKERS = _NUM_CORES * _NUM_SUBCORES

_CHUNK = 128
_NBUF = 5
_OUTW = 128
_TBLK = 8192


def _transpose_step(t_ref, i_ref, out_ref):
    out_ref[...] = lax.dot_general(
        t_ref[...],
        i_ref[...],
        (((0,), (0,)), ((), ())),
        precision=lax.Precision.HIGHEST,
        preferred_element_type=jnp.float32,
    )


def _make_tc_repack(vocab: int, emb: int):
    grid = (vocab + _TBLK - 1) // _TBLK
    return pl.pallas_call(
        _transpose_step,
        grid=(grid,),
        in_specs=[
            pl.BlockSpec((emb, _TBLK), lambda i: (0, i)),
            pl.BlockSpec((emb, _OUTW), lambda i: (0, 0)),
        ],
        out_specs=pl.BlockSpec((_TBLK, _OUTW), lambda i: (i, 0)),
        out_shape=jax.ShapeDtypeStruct((vocab, _OUTW), jnp.float32),
    )


def _make_sc_gather(n_idx: int):
    assert n_idx % (_NUM_WORKERS * _CHUNK * _NBUF) == 0
    per_w = n_idx // _NUM_WORKERS
    groups = per_w // (_CHUNK * _NBUF)

    mesh = plsc.VectorSubcoreMesh(core_axis_name="c", subcore_axis_name="s")

    @functools.partial(
        pl.kernel,
        mesh=mesh,
        out_type=jax.ShapeDtypeStruct((n_idx, _OUTW), jnp.float32),
        compiler_params=pltpu.CompilerParams(use_tc_tiling_on_sc=False),
        scratch_types=[
            pltpu.VMEM((per_w,), jnp.int32),
            [pltpu.VMEM((_CHUNK, _OUTW), jnp.float32) for _ in range(_NBUF)],
            [pltpu.SemaphoreType.DMA for _ in range(_NBUF)],
        ],
    )
    def gather_kernel(table_hbm, idx_hbm, out_hbm, idx_v, rows, sems):
        wid = lax.axis_index("s") * _NUM_CORES + lax.axis_index("c")
        base = wid * per_w
        pltpu.sync_copy(idx_hbm.at[pl.ds(base, per_w)], idx_v)

        def group_body(g, carry):
            goff = g * (_CHUNK * _NBUF)
            copies = []
            for b in range(_NBUF):
                off = goff + b * _CHUNK
                copies.append(
                    pltpu.async_copy(
                        table_hbm.at[idx_v.at[pl.ds(off, _CHUNK)]],
                        rows[b],
                        sems[b],
                    )
                )
            for b in range(_NBUF):
                off = goff + b * _CHUNK
                copies[b].wait()
                pltpu.sync_copy(rows[b], out_hbm.at[pl.ds(base + off, _CHUNK)])
            return carry

        lax.fori_loop(0, groups, group_body, 0)

    return gather_kernel


def _rnn_step(emb_ref, wih_ref, whh_ref, b_ref, out_ref, embout_ref, h_ref):
    t = pl.program_id(0)

    @pl.when(t == 0)
    def _():
        h_ref[...] = jnp.zeros_like(h_ref)

    hid = out_ref.shape[2]
    x = emb_ref[0][:, :hid]
    embout_ref[0] = x
    h = h_ref[...]
    pre = (
        jnp.dot(x, wih_ref[...], preferred_element_type=jnp.float32)
        + jnp.dot(h, whh_ref[...], preferred_element_type=jnp.float32)
        + b_ref[...]
    )
    h_new = jnp.tanh(pre)
    h_ref[...] = h_new
    out_ref[0] = h_new


def _make_tc_rnn(seq: int, batch: int, emb: int, hid: int):
    return pl.pallas_call(
        _rnn_step,
        grid=(seq,),
        in_specs=[
            pl.BlockSpec((1, batch, _OUTW), lambda t: (t, 0, 0)),
            pl.BlockSpec((emb, hid), lambda t: (0, 0)),
            pl.BlockSpec((hid, hid), lambda t: (0, 0)),
            pl.BlockSpec((1, hid), lambda t: (0, 0)),
        ],
        out_specs=[
            pl.BlockSpec((1, batch, hid), lambda t: (t, 0, 0)),
            pl.BlockSpec((1, batch, emb), lambda t: (t, 0, 0)),
        ],
        out_shape=[
            jax.ShapeDtypeStruct((seq, batch, hid), jnp.float32),
            jax.ShapeDtypeStruct((seq, batch, emb), jnp.float32),
        ],
        scratch_shapes=[pltpu.VMEM((batch, hid), jnp.float32)],
    )


def kernel(input_seq, emb_table, W_ih, W_hh, b_ih, b_hh):
    seq, batch = input_seq.shape
    vocab, emb = emb_table.shape
    hid = W_hh.shape[0]

    idx_flat = input_seq.reshape(-1).astype(jnp.int32)
    ipad = jnp.eye(emb, _OUTW, dtype=jnp.float32)
    table128 = _make_tc_repack(vocab, emb)(emb_table.T, ipad)
    gathered = _make_sc_gather(seq * batch)(table128, idx_flat)
    emb_wide = gathered.reshape(seq, batch, _OUTW)

    bias = (b_ih + b_hh).reshape(1, hid)
    output_seq, embedded_seq = _make_tc_rnn(seq, batch, emb, hid)(
        emb_wide, W_ih.T, W_hh.T, bias
    )
    last_hidden = output_seq[seq - 1 : seq]
    return output_seq, last_hidden, embedded_seq


# speedup vs baseline: 1.4198x; 1.2478x over previous
import functools

import jax
import jax.numpy as jnp
from jax import lax
from jax.experimental import pallas as pl
from jax.experimental.pallas import tpu as pltpu
from jax.experimental.pallas import tpu_sc as plsc

_NUM_CORES = 2
_NUM_SUBCORES = 16
_NUM_WORKERS = _NUM_CORES * _NUM_SUBCORES

_CHUNK = 128
_NBUF = 5
_OUTW = 128
_TBLK = 8192


def _transpose_step(t_ref, i_ref, out_ref):
    out_ref[...] = lax.dot_general(
        t_ref[...],
        i_ref[...],
        (((0,), (0,)), ((), ())),
        precision=lax.Precision.HIGHEST,
        preferred_element_type=jnp.float32,
    )


def _make_tc_repack(vocab: int, emb: int):
    grid = (vocab + _TBLK - 1) // _TBLK
    return pl.pallas_call(
        _transpose_step,
        grid=(grid,),
        in_specs=[
            pl.BlockSpec((emb, _TBLK), lambda i: (0, i)),
            pl.BlockSpec((emb, _OUTW), lambda i: (0, 0)),
        ],
        out_specs=pl.BlockSpec((_TBLK, _OUTW), lambda i: (i, 0)),
        out_shape=jax.ShapeDtypeStruct((vocab, _OUTW), jnp.float32),
    )


def _make_sc_gather(n_idx: int):
    assert n_idx % (_NUM_WORKERS * _CHUNK * _NBUF) == 0
    per_w = n_idx // _NUM_WORKERS
    groups = per_w // (_CHUNK * _NBUF)

    mesh = plsc.VectorSubcoreMesh(core_axis_name="c", subcore_axis_name="s")

    @functools.partial(
        pl.kernel,
        mesh=mesh,
        out_type=jax.ShapeDtypeStruct((n_idx, _OUTW), jnp.float32),
        compiler_params=pltpu.CompilerParams(use_tc_tiling_on_sc=False),
        scratch_types=[
            pltpu.VMEM((per_w,), jnp.int32),
            [pltpu.VMEM((_CHUNK, _OUTW), jnp.float32) for _ in range(_NBUF)],
            [pltpu.SemaphoreType.DMA for _ in range(_NBUF)],
        ],
    )
    def gather_kernel(table_hbm, idx_hbm, out_hbm, idx_v, rows, sems):
        wid = lax.axis_index("s") * _NUM_CORES + lax.axis_index("c")
        base = wid * per_w
        pltpu.sync_copy(idx_hbm.at[pl.ds(base, per_w)], idx_v)

        def group_body(g, carry):
            goff = g * (_CHUNK * _NBUF)
            copies = []
            for b in range(_NBUF):
                off = goff + b * _CHUNK
                copies.append(
                    pltpu.async_copy(
                        table_hbm.at[idx_v.at[pl.ds(off, _CHUNK)]],
                        rows[b],
                        sems[b],
                    )
                )
            for b in range(_NBUF):
                off = goff + b * _CHUNK
                copies[b].wait()
                pltpu.sync_copy(rows[b], out_hbm.at[pl.ds(base + off, _CHUNK)])
            return carry

        lax.fori_loop(0, groups, group_body, 0)

    return gather_kernel


def _rnn_step(emb_ref, wih_ref, whh_ref, b_ref, out_ref, embout_ref, h_ref):
    t = pl.program_id(0)

    @pl.when(t == 0)
    def _():
        h_ref[...] = jnp.zeros_like(h_ref)

    hid = out_ref.shape[1]
    xt = jnp.swapaxes(emb_ref[0][:, :hid], 0, 1)
    embout_ref[0] = xt
    h = h_ref[...]
    pre = (
        jnp.dot(wih_ref[...], xt, preferred_element_type=jnp.float32)
        + jnp.dot(whh_ref[...], h, preferred_element_type=jnp.float32)
        + b_ref[...]
    )
    h_new = jnp.tanh(pre)
    h_ref[...] = h_new
    out_ref[0] = h_new


def _make_tc_rnn(seq: int, batch: int, emb: int, hid: int):
    return pl.pallas_call(
        _rnn_step,
        grid=(seq,),
        in_specs=[
            pl.BlockSpec((1, batch, _OUTW), lambda t: (t, 0, 0)),
            pl.BlockSpec((hid, emb), lambda t: (0, 0)),
            pl.BlockSpec((hid, hid), lambda t: (0, 0)),
            pl.BlockSpec((hid, 1), lambda t: (0, 0)),
        ],
        out_specs=[
            pl.BlockSpec((1, hid, batch), lambda t: (t, 0, 0)),
            pl.BlockSpec((1, emb, batch), lambda t: (t, 0, 0)),
        ],
        out_shape=[
            jax.ShapeDtypeStruct((seq, hid, batch), jnp.float32),
            jax.ShapeDtypeStruct((seq, emb, batch), jnp.float32),
        ],
        scratch_shapes=[pltpu.VMEM((hid, batch), jnp.float32)],
    )


def kernel(input_seq, emb_table, W_ih, W_hh, b_ih, b_hh):
    seq, batch = input_seq.shape
    vocab, emb = emb_table.shape
    hid = W_hh.shape[0]

    idx_flat = input_seq.reshape(-1).astype(jnp.int32)
    ipad = jnp.eye(emb, _OUTW, dtype=jnp.float32)
    table128 = _make_tc_repack(vocab, emb)(emb_table.T, ipad)
    gathered = _make_sc_gather(seq * batch)(table128, idx_flat)
    emb_wide = gathered.reshape(seq, batch, _OUTW)

    bias = (b_ih + b_hh).reshape(hid, 1)
    out_t, emb_t = _make_tc_rnn(seq, batch, emb, hid)(
        emb_wide, W_ih, W_hh, bias
    )
    output_seq = out_t.transpose(0, 2, 1)
    embedded_seq = emb_t.transpose(0, 2, 1)
    last_hidden = output_seq[seq - 1 : seq]
    return output_seq, last_hidden, embedded_seq


# speedup vs baseline: 1.8737x; 1.3197x over previous
import functools

import jax
import jax.numpy as jnp
from jax import lax
from jax.experimental import pallas as pl
from jax.experimental.pallas import tpu as pltpu
from jax.experimental.pallas import tpu_sc as plsc

_NUM_CORES = 2
_NUM_SUBCORES = 16
_NUM_WORKERS = _NUM_CORES * _NUM_SUBCORES

_CHUNK = 128
_NBUF = 5
_OUTW = 128
_TBLK = 8192


def _transpose_step(t_ref, out_ref):
    xt = jnp.swapaxes(t_ref[...], 0, 1)
    out_ref[...] = jnp.concatenate([xt, jnp.zeros_like(xt)], axis=1)


def _make_tc_repack(vocab: int, emb: int):
    grid = (vocab + _TBLK - 1) // _TBLK
    return pl.pallas_call(
        _transpose_step,
        grid=(grid,),
        in_specs=[
            pl.BlockSpec((emb, _TBLK), lambda i: (0, i)),
        ],
        out_specs=pl.BlockSpec((_TBLK, _OUTW), lambda i: (i, 0)),
        out_shape=jax.ShapeDtypeStruct((vocab, _OUTW), jnp.float32),
    )


def _make_sc_gather(n_idx: int):
    assert n_idx % (_NUM_WORKERS * _CHUNK * _NBUF) == 0
    per_w = n_idx // _NUM_WORKERS
    groups = per_w // (_CHUNK * _NBUF)

    mesh = plsc.VectorSubcoreMesh(core_axis_name="c", subcore_axis_name="s")

    @functools.partial(
        pl.kernel,
        mesh=mesh,
        out_type=jax.ShapeDtypeStruct((n_idx, _OUTW), jnp.float32),
        compiler_params=pltpu.CompilerParams(use_tc_tiling_on_sc=False),
        scratch_types=[
            pltpu.VMEM((per_w,), jnp.int32),
            [pltpu.VMEM((_CHUNK, _OUTW), jnp.float32) for _ in range(_NBUF)],
            [pltpu.SemaphoreType.DMA for _ in range(_NBUF)],
        ],
    )
    def gather_kernel(table_hbm, idx_hbm, out_hbm, idx_v, rows, sems):
        wid = lax.axis_index("s") * _NUM_CORES + lax.axis_index("c")
        base = wid * per_w
        pltpu.sync_copy(idx_hbm.at[pl.ds(base, per_w)], idx_v)

        def group_body(g, carry):
            goff = g * (_CHUNK * _NBUF)
            copies = []
            for b in range(_NBUF):
                off = goff + b * _CHUNK
                copies.append(
                    pltpu.async_copy(
                        table_hbm.at[idx_v.at[pl.ds(off, _CHUNK)]],
                        rows[b],
                        sems[b],
                    )
                )
            for b in range(_NBUF):
                off = goff + b * _CHUNK
                copies[b].wait()
                pltpu.sync_copy(rows[b], out_hbm.at[pl.ds(base + off, _CHUNK)])
            return carry

        lax.fori_loop(0, groups, group_body, 0)

    return gather_kernel


def _rnn_step(emb_ref, wih_ref, whh_ref, b_ref, out_ref, embout_ref, h_ref):
    t = pl.program_id(0)

    @pl.when(t == 0)
    def _():
        h_ref[...] = jnp.zeros_like(h_ref)

    hid = out_ref.shape[1]
    xt = jnp.swapaxes(emb_ref[0][:, :hid], 0, 1)
    embout_ref[0] = xt
    h = h_ref[...]
    pre = (
        jnp.dot(wih_ref[...], xt, preferred_element_type=jnp.float32)
        + jnp.dot(whh_ref[...], h, preferred_element_type=jnp.float32)
        + b_ref[...]
    )
    h_new = jnp.tanh(pre)
    h_ref[...] = h_new
    out_ref[0] = h_new


def _make_tc_rnn(seq: int, batch: int, emb: int, hid: int):
    return pl.pallas_call(
        _rnn_step,
        grid=(seq,),
        in_specs=[
            pl.BlockSpec((1, batch, _OUTW), lambda t: (t, 0, 0)),
            pl.BlockSpec((hid, emb), lambda t: (0, 0)),
            pl.BlockSpec((hid, hid), lambda t: (0, 0)),
            pl.BlockSpec((hid, 1), lambda t: (0, 0)),
        ],
        out_specs=[
            pl.BlockSpec((1, hid, batch), lambda t: (t, 0, 0)),
            pl.BlockSpec((1, emb, batch), lambda t: (t, 0, 0)),
        ],
        out_shape=[
            jax.ShapeDtypeStruct((seq, hid, batch), jnp.float32),
            jax.ShapeDtypeStruct((seq, emb, batch), jnp.float32),
        ],
        scratch_shapes=[pltpu.VMEM((hid, batch), jnp.float32)],
    )


def kernel(input_seq, emb_table, W_ih, W_hh, b_ih, b_hh):
    seq, batch = input_seq.shape
    vocab, emb = emb_table.shape
    hid = W_hh.shape[0]

    idx_flat = input_seq.reshape(-1).astype(jnp.int32)
    table128 = _make_tc_repack(vocab, emb)(emb_table.T)
    gathered = _make_sc_gather(seq * batch)(table128, idx_flat)
    emb_wide = gathered.reshape(seq, batch, _OUTW)

    bias = (b_ih + b_hh).reshape(hid, 1)
    out_t, emb_t = _make_tc_rnn(seq, batch, emb, hid)(
        emb_wide, W_ih, W_hh, bias
    )
    output_seq = out_t.transpose(0, 2, 1)
    embedded_seq = emb_t.transpose(0, 2, 1)
    last_hidden = output_seq[seq - 1 : seq]
    return output_seq, last_hidden, embedded_seq


# speedup vs baseline: 2.2668x; 1.2098x over previous
import functools

import jax
import jax.numpy as jnp
from jax import lax
from jax.experimental import pallas as pl
from jax.experimental.pallas import tpu as pltpu
from jax.experimental.pallas import tpu_sc as plsc

_NUM_CORES = 2
_NUM_SUBCORES = 16
_NUM_WORKERS = _NUM_CORES * _NUM_SUBCORES

_CHUNK = 128
_NBUF = 5
_OUTW = 128
_TBLK = 8192


def _transpose_step(t_ref, out_ref):
    xt = jnp.swapaxes(t_ref[...], 0, 1)
    out_ref[...] = jnp.concatenate([xt, jnp.zeros_like(xt)], axis=1)


def _make_tc_repack(vocab: int, emb: int):
    grid = (vocab + _TBLK - 1) // _TBLK
    return pl.pallas_call(
        _transpose_step,
        grid=(grid,),
        in_specs=[
            pl.BlockSpec((emb, _TBLK), lambda i: (0, i)),
        ],
        out_specs=pl.BlockSpec((_TBLK, _OUTW), lambda i: (i, 0)),
        out_shape=jax.ShapeDtypeStruct((vocab, _OUTW), jnp.float32),
    )


def _make_sc_gather(n_idx: int):
    assert n_idx % (_NUM_WORKERS * _CHUNK * _NBUF) == 0
    per_w = n_idx // _NUM_WORKERS
    groups = per_w // (_CHUNK * _NBUF)

    mesh = plsc.VectorSubcoreMesh(core_axis_name="c", subcore_axis_name="s")

    @functools.partial(
        pl.kernel,
        mesh=mesh,
        out_type=jax.ShapeDtypeStruct((n_idx, _OUTW), jnp.float32),
        compiler_params=pltpu.CompilerParams(use_tc_tiling_on_sc=False),
        scratch_types=[
            pltpu.VMEM((per_w,), jnp.int32),
            [pltpu.VMEM((_CHUNK, _OUTW), jnp.float32) for _ in range(_NBUF)],
            [pltpu.SemaphoreType.DMA for _ in range(_NBUF)],
        ],
    )
    def gather_kernel(table_hbm, idx_hbm, out_hbm, idx_v, rows, sems):
        wid = lax.axis_index("s") * _NUM_CORES + lax.axis_index("c")
        base = wid * per_w
        pltpu.sync_copy(idx_hbm.at[pl.ds(base, per_w)], idx_v)

        def group_body(g, carry):
            goff = g * (_CHUNK * _NBUF)
            copies = []
            for b in range(_NBUF):
                off = goff + b * _CHUNK
                copies.append(
                    pltpu.async_copy(
                        table_hbm.at[idx_v.at[pl.ds(off, _CHUNK)]],
                        rows[b],
                        sems[b],
                    )
                )
            for b in range(_NBUF):
                off = goff + b * _CHUNK
                copies[b].wait()
                pltpu.sync_copy(rows[b], out_hbm.at[pl.ds(base + off, _CHUNK)])
            return carry

        lax.fori_loop(0, groups, group_body, 0)

    return gather_kernel


_KS = 5


def _rnn_step(emb_ref, wih_ref, whh_ref, b_ref, out_ref, embout_ref, h_ref):
    t = pl.program_id(0)

    @pl.when(t == 0)
    def _():
        h_ref[...] = jnp.zeros_like(h_ref)

    hid = out_ref.shape[1]
    wih = wih_ref[...]
    whh = whh_ref[...]
    b = b_ref[...]
    h = h_ref[...]
    for s in range(_KS):
        xt = jnp.swapaxes(emb_ref[s][:, :hid], 0, 1)
        embout_ref[s] = xt
        pre = (
            jnp.dot(wih, xt, preferred_element_type=jnp.float32)
            + jnp.dot(whh, h, preferred_element_type=jnp.float32)
            + b
        )
        h = jnp.tanh(pre)
        out_ref[s] = h
    h_ref[...] = h


def _make_tc_rnn(seq: int, batch: int, emb: int, hid: int):
    assert seq % _KS == 0
    return pl.pallas_call(
        _rnn_step,
        grid=(seq // _KS,),
        in_specs=[
            pl.BlockSpec((_KS, batch, _OUTW), lambda t: (t, 0, 0)),
            pl.BlockSpec((hid, emb), lambda t: (0, 0)),
            pl.BlockSpec((hid, hid), lambda t: (0, 0)),
            pl.BlockSpec((hid, 1), lambda t: (0, 0)),
        ],
        out_specs=[
            pl.BlockSpec((_KS, hid, batch), lambda t: (t, 0, 0)),
            pl.BlockSpec((_KS, emb, batch), lambda t: (t, 0, 0)),
        ],
        out_shape=[
            jax.ShapeDtypeStruct((seq, hid, batch), jnp.float32),
            jax.ShapeDtypeStruct((seq, emb, batch), jnp.float32),
        ],
        scratch_shapes=[pltpu.VMEM((hid, batch), jnp.float32)],
    )


def kernel(input_seq, emb_table, W_ih, W_hh, b_ih, b_hh):
    seq, batch = input_seq.shape
    vocab, emb = emb_table.shape
    hid = W_hh.shape[0]

    idx_flat = input_seq.reshape(-1).astype(jnp.int32)
    table128 = _make_tc_repack(vocab, emb)(emb_table.T)
    gathered = _make_sc_gather(seq * batch)(table128, idx_flat)
    emb_wide = gathered.reshape(seq, batch, _OUTW)

    bias = (b_ih + b_hh).reshape(hid, 1)
    out_t, emb_t = _make_tc_rnn(seq, batch, emb, hid)(
        emb_wide, W_ih, W_hh, bias
    )
    output_seq = out_t.transpose(0, 2, 1)
    embedded_seq = emb_t.transpose(0, 2, 1)
    last_hidden = output_seq[seq - 1 : seq]
    return output_seq, last_hidden, embedded_seq


# speedup vs baseline: 2.3710x; 1.0460x over previous
import functools

import jax
import jax.numpy as jnp
from jax import lax
from jax.experimental import pallas as pl
from jax.experimental.pallas import tpu as pltpu
from jax.experimental.pallas import tpu_sc as plsc

_NUM_CORES = 2
_NUM_SUBCORES = 16
_NUM_WORKERS = _NUM_CORES * _NUM_SUBCORES

_CHUNK = 128
_NBUF = 5
_OUTW = 128
_TBLK = 16384


def _transpose_step(t_ref, out_ref):
    xt = jnp.swapaxes(t_ref[...], 0, 1)
    out_ref[...] = jnp.concatenate([xt, jnp.zeros_like(xt)], axis=1)


def _make_tc_repack(vocab: int, emb: int):
    grid = (vocab + _TBLK - 1) // _TBLK
    return pl.pallas_call(
        _transpose_step,
        grid=(grid,),
        in_specs=[
            pl.BlockSpec((emb, _TBLK), lambda i: (0, i)),
        ],
        out_specs=pl.BlockSpec((_TBLK, _OUTW), lambda i: (i, 0)),
        out_shape=jax.ShapeDtypeStruct((vocab, _OUTW), jnp.float32),
    )


def _make_sc_gather(n_idx: int):
    assert n_idx % (_NUM_WORKERS * _CHUNK * _NBUF) == 0
    per_w = n_idx // _NUM_WORKERS
    groups = per_w // (_CHUNK * _NBUF)

    mesh = plsc.VectorSubcoreMesh(core_axis_name="c", subcore_axis_name="s")

    @functools.partial(
        pl.kernel,
        mesh=mesh,
        out_type=jax.ShapeDtypeStruct((n_idx, _OUTW), jnp.float32),
        compiler_params=pltpu.CompilerParams(use_tc_tiling_on_sc=False),
        scratch_types=[
            pltpu.VMEM((per_w,), jnp.int32),
            [pltpu.VMEM((_CHUNK, _OUTW), jnp.float32) for _ in range(_NBUF)],
            [pltpu.SemaphoreType.DMA for _ in range(_NBUF)],
        ],
    )
    def gather_kernel(table_hbm, idx_hbm, out_hbm, idx_v, rows, sems):
        wid = lax.axis_index("s") * _NUM_CORES + lax.axis_index("c")
        base = wid * per_w
        pltpu.sync_copy(idx_hbm.at[pl.ds(base, per_w)], idx_v)

        def group_body(g, carry):
            goff = g * (_CHUNK * _NBUF)
            copies = []
            for b in range(_NBUF):
                off = goff + b * _CHUNK
                copies.append(
                    pltpu.async_copy(
                        table_hbm.at[idx_v.at[pl.ds(off, _CHUNK)]],
                        rows[b],
                        sems[b],
                    )
                )
            for b in range(_NBUF):
                off = goff + b * _CHUNK
                copies[b].wait()
                pltpu.sync_copy(rows[b], out_hbm.at[pl.ds(base + off, _CHUNK)])
            return carry

        lax.fori_loop(0, groups, group_body, 0)

    return gather_kernel


_KS = 5


def _rnn_step(emb_ref, wih_ref, whh_ref, b_ref, out_ref, embout_ref, h_ref):
    t = pl.program_id(0)

    @pl.when(t == 0)
    def _():
        h_ref[...] = jnp.zeros_like(h_ref)

    hid = out_ref.shape[1]
    wih = wih_ref[...]
    whh = whh_ref[...]
    b = b_ref[...]
    h = h_ref[...]
    for s in range(_KS):
        xt = jnp.swapaxes(emb_ref[s][:, :hid], 0, 1)
        embout_ref[s] = xt
        pre = (
            jnp.dot(wih, xt, preferred_element_type=jnp.float32)
            + jnp.dot(whh, h, preferred_element_type=jnp.float32)
            + b
        )
        h = jnp.tanh(pre)
        out_ref[s] = h
    h_ref[...] = h


def _make_tc_rnn(seq: int, batch: int, emb: int, hid: int):
    assert seq % _KS == 0
    return pl.pallas_call(
        _rnn_step,
        grid=(seq // _KS,),
        in_specs=[
            pl.BlockSpec((_KS, batch, _OUTW), lambda t: (t, 0, 0)),
            pl.BlockSpec((hid, emb), lambda t: (0, 0)),
            pl.BlockSpec((hid, hid), lambda t: (0, 0)),
            pl.BlockSpec((hid, 1), lambda t: (0, 0)),
        ],
        out_specs=[
            pl.BlockSpec((_KS, hid, batch), lambda t: (t, 0, 0)),
            pl.BlockSpec((_KS, emb, batch), lambda t: (t, 0, 0)),
        ],
        out_shape=[
            jax.ShapeDtypeStruct((seq, hid, batch), jnp.float32),
            jax.ShapeDtypeStruct((seq, emb, batch), jnp.float32),
        ],
        scratch_shapes=[pltpu.VMEM((hid, batch), jnp.float32)],
    )


def kernel(input_seq, emb_table, W_ih, W_hh, b_ih, b_hh):
    seq, batch = input_seq.shape
    vocab, emb = emb_table.shape
    hid = W_hh.shape[0]

    idx_flat = input_seq.reshape(-1).astype(jnp.int32)
    table128 = _make_tc_repack(vocab, emb)(emb_table.T)
    gathered = _make_sc_gather(seq * batch)(table128, idx_flat)
    emb_wide = gathered.reshape(seq, batch, _OUTW)

    bias = (b_ih + b_hh).reshape(hid, 1)
    out_t, emb_t = _make_tc_rnn(seq, batch, emb, hid)(
        emb_wide, W_ih, W_hh, bias
    )
    output_seq = out_t.transpose(0, 2, 1)
    embedded_seq = emb_t.transpose(0, 2, 1)
    last_hidden = output_seq[seq - 1 : seq]
    return output_seq, last_hidden, embedded_seq


# speedup vs baseline: 2.4053x; 1.0145x over previous
import functools

import jax
import jax.numpy as jnp
from jax import lax
from jax.experimental import pallas as pl
from jax.experimental.pallas import tpu as pltpu
from jax.experimental.pallas import tpu_sc as plsc

_NUM_CORES = 2
_NUM_SUBCORES = 16
_NUM_WORKERS = _NUM_CORES * _NUM_SUBCORES

_CHUNK = 128
_NBUF = 5
_OUTW = 128
_TBLK = 32768


def _transpose_step(t_ref, out_ref):
    xt = jnp.swapaxes(t_ref[...], 0, 1)
    out_ref[...] = jnp.concatenate([xt, jnp.zeros_like(xt)], axis=1)


def _make_tc_repack(vocab: int, emb: int):
    grid = (vocab + _TBLK - 1) // _TBLK
    return pl.pallas_call(
        _transpose_step,
        grid=(grid,),
        in_specs=[
            pl.BlockSpec((emb, _TBLK), lambda i: (0, i)),
        ],
        out_specs=pl.BlockSpec((_TBLK, _OUTW), lambda i: (i, 0)),
        out_shape=jax.ShapeDtypeStruct((vocab, _OUTW), jnp.float32),
    )


def _make_sc_gather(n_idx: int):
    assert n_idx % (_NUM_WORKERS * _CHUNK * _NBUF) == 0
    per_w = n_idx // _NUM_WORKERS
    groups = per_w // (_CHUNK * _NBUF)

    mesh = plsc.VectorSubcoreMesh(core_axis_name="c", subcore_axis_name="s")

    @functools.partial(
        pl.kernel,
        mesh=mesh,
        out_type=jax.ShapeDtypeStruct((n_idx, _OUTW), jnp.float32),
        compiler_params=pltpu.CompilerParams(use_tc_tiling_on_sc=False),
        scratch_types=[
            pltpu.VMEM((per_w,), jnp.int32),
            [pltpu.VMEM((_CHUNK, _OUTW), jnp.float32) for _ in range(_NBUF)],
            [pltpu.SemaphoreType.DMA for _ in range(_NBUF)],
        ],
    )
    def gather_kernel(table_hbm, idx_hbm, out_hbm, idx_v, rows, sems):
        wid = lax.axis_index("s") * _NUM_CORES + lax.axis_index("c")
        base = wid * per_w
        pltpu.sync_copy(idx_hbm.at[pl.ds(base, per_w)], idx_v)

        def group_body(g, carry):
            goff = g * (_CHUNK * _NBUF)
            copies = []
            for b in range(_NBUF):
                off = goff + b * _CHUNK
                copies.append(
                    pltpu.async_copy(
                        table_hbm.at[idx_v.at[pl.ds(off, _CHUNK)]],
                        rows[b],
                        sems[b],
                    )
                )
            for b in range(_NBUF):
                off = goff + b * _CHUNK
                copies[b].wait()
                pltpu.sync_copy(rows[b], out_hbm.at[pl.ds(base + off, _CHUNK)])
            return carry

        lax.fori_loop(0, groups, group_body, 0)

    return gather_kernel


_KS = 5


def _rnn_step(emb_ref, wih_ref, whh_ref, b_ref, out_ref, embout_ref, h_ref):
    t = pl.program_id(0)

    @pl.when(t == 0)
    def _():
        h_ref[...] = jnp.zeros_like(h_ref)

    hid = out_ref.shape[1]
    wih = wih_ref[...]
    whh = whh_ref[...]
    b = b_ref[...]
    h = h_ref[...]
    for s in range(_KS):
        xt = jnp.swapaxes(emb_ref[s][:, :hid], 0, 1)
        embout_ref[s] = xt
        pre = (
            jnp.dot(wih, xt, preferred_element_type=jnp.float32)
            + jnp.dot(whh, h, preferred_element_type=jnp.float32)
            + b
        )
        h = jnp.tanh(pre)
        out_ref[s] = h
    h_ref[...] = h


def _make_tc_rnn(seq: int, batch: int, emb: int, hid: int):
    assert seq % _KS == 0
    return pl.pallas_call(
        _rnn_step,
        grid=(seq // _KS,),
        in_specs=[
            pl.BlockSpec((_KS, batch, _OUTW), lambda t: (t, 0, 0)),
            pl.BlockSpec((hid, emb), lambda t: (0, 0)),
            pl.BlockSpec((hid, hid), lambda t: (0, 0)),
            pl.BlockSpec((hid, 1), lambda t: (0, 0)),
        ],
        out_specs=[
            pl.BlockSpec((_KS, hid, batch), lambda t: (t, 0, 0)),
            pl.BlockSpec((_KS, emb, batch), lambda t: (t, 0, 0)),
        ],
        out_shape=[
            jax.ShapeDtypeStruct((seq, hid, batch), jnp.float32),
            jax.ShapeDtypeStruct((seq, emb, batch), jnp.float32),
        ],
        scratch_shapes=[pltpu.VMEM((hid, batch), jnp.float32)],
    )


def kernel(input_seq, emb_table, W_ih, W_hh, b_ih, b_hh):
    seq, batch = input_seq.shape
    vocab, emb = emb_table.shape
    hid = W_hh.shape[0]

    idx_flat = input_seq.reshape(-1).astype(jnp.int32)
    table128 = _make_tc_repack(vocab, emb)(emb_table.T)
    gathered = _make_sc_gather(seq * batch)(table128, idx_flat)
    emb_wide = gathered.reshape(seq, batch, _OUTW)

    bias = (b_ih + b_hh).reshape(hid, 1)
    out_t, emb_t = _make_tc_rnn(seq, batch, emb, hid)(
        emb_wide, W_ih, W_hh, bias
    )
    output_seq = out_t.transpose(0, 2, 1)
    embedded_seq = emb_t.transpose(0, 2, 1)
    last_hidden = output_seq[seq - 1 : seq]
    return output_seq, last_hidden, embedded_seq


# speedup vs baseline: 2.4068x; 1.0006x over previous
import functools

import jax
import jax.numpy as jnp
from jax import lax
from jax.experimental import pallas as pl
from jax.experimental.pallas import tpu as pltpu
from jax.experimental.pallas import tpu_sc as plsc

_NUM_CORES = 2
_NUM_SUBCORES = 16
_NUM_WORKERS = _NUM_CORES * _NUM_SUBCORES

_CHUNK = 128
_NBUF = 5
_OUTW = 128
_TBLK = 32768


def _transpose_step(t_ref, out_ref):
    xt = jnp.swapaxes(t_ref[...], 0, 1)
    out_ref[...] = jnp.concatenate([xt, jnp.zeros_like(xt)], axis=1)


def _make_tc_repack(vocab: int, emb: int):
    grid = (vocab + _TBLK - 1) // _TBLK
    return pl.pallas_call(
        _transpose_step,
        grid=(grid,),
        in_specs=[
            pl.BlockSpec((emb, _TBLK), lambda i: (0, i)),
        ],
        out_specs=pl.BlockSpec((_TBLK, _OUTW), lambda i: (i, 0)),
        out_shape=jax.ShapeDtypeStruct((vocab, _OUTW), jnp.float32),
    )


def _make_sc_gather(n_idx: int, vocab2: int, emb: int):
    assert n_idx % (_NUM_WORKERS * _CHUNK * _NBUF) == 0
    per_w = n_idx // _NUM_WORKERS
    groups = per_w // (_CHUNK * _NBUF)

    mesh = plsc.VectorSubcoreMesh(core_axis_name="c", subcore_axis_name="s")

    @functools.partial(
        pl.kernel,
        mesh=mesh,
        out_type=jax.ShapeDtypeStruct((n_idx, _OUTW), jnp.float32),
        compiler_params=pltpu.CompilerParams(use_tc_tiling_on_sc=False),
        scratch_types=[
            pltpu.VMEM((per_w,), jnp.int32),
            [pltpu.VMEM((_CHUNK, _OUTW), jnp.float32) for _ in range(_NBUF)],
            [pltpu.SemaphoreType.DMA for _ in range(_NBUF)],
        ],
    )
    def gather_kernel(table_hbm, idx_hbm, out_hbm, idx_v, rows, sems):
        wid = lax.axis_index("s") * _NUM_CORES + lax.axis_index("c")
        base = wid * per_w
        pltpu.sync_copy(idx_hbm.at[pl.ds(base, per_w)], idx_v)

        def group_body(g, carry):
            goff = g * (_CHUNK * _NBUF)
            copies = []
            for b in range(_NBUF):
                off = goff + b * _CHUNK
                copies.append(
                    pltpu.async_copy(
                        table_hbm.at[idx_v.at[pl.ds(off, _CHUNK)]],
                        rows[b],
                        sems[b],
                    )
                )
            for b in range(_NBUF):
                off = goff + b * _CHUNK
                copies[b].wait()
                pltpu.sync_copy(
                    rows[b].at[:, pl.ds(0, emb)],
                    out_hbm.at[pl.ds(base + off, _CHUNK), pl.ds(0, emb)],
                )
            return carry

        lax.fori_loop(0, groups, group_body, 0)

    return gather_kernel


_KS = 5


def _rnn_step(emb_ref, wih_ref, whh_ref, b_ref, out_ref, embout_ref, h_ref):
    t = pl.program_id(0)

    @pl.when(t == 0)
    def _():
        h_ref[...] = jnp.zeros_like(h_ref)

    hid = out_ref.shape[1]
    wih = wih_ref[...]
    whh = whh_ref[...]
    b = b_ref[...]
    h = h_ref[...]
    for s in range(_KS):
        xt = jnp.swapaxes(emb_ref[s][:, :hid], 0, 1)
        embout_ref[s] = xt
        pre = (
            jnp.dot(wih, xt, preferred_element_type=jnp.float32)
            + jnp.dot(whh, h, preferred_element_type=jnp.float32)
            + b
        )
        h = jnp.tanh(pre)
        out_ref[s] = h
    h_ref[...] = h


def _make_tc_rnn(seq: int, batch: int, emb: int, hid: int):
    assert seq % _KS == 0
    return pl.pallas_call(
        _rnn_step,
        grid=(seq // _KS,),
        in_specs=[
            pl.BlockSpec((_KS, batch, _OUTW), lambda t: (t, 0, 0)),
            pl.BlockSpec((hid, emb), lambda t: (0, 0)),
            pl.BlockSpec((hid, hid), lambda t: (0, 0)),
            pl.BlockSpec((hid, 1), lambda t: (0, 0)),
        ],
        out_specs=[
            pl.BlockSpec((_KS, hid, batch), lambda t: (t, 0, 0)),
            pl.BlockSpec((_KS, emb, batch), lambda t: (t, 0, 0)),
        ],
        out_shape=[
            jax.ShapeDtypeStruct((seq, hid, batch), jnp.float32),
            jax.ShapeDtypeStruct((seq, emb, batch), jnp.float32),
        ],
        scratch_shapes=[pltpu.VMEM((hid, batch), jnp.float32)],
    )


def kernel(input_seq, emb_table, W_ih, W_hh, b_ih, b_hh):
    seq, batch = input_seq.shape
    vocab, emb = emb_table.shape
    hid = W_hh.shape[0]

    idx_flat = input_seq.reshape(-1).astype(jnp.int32)
    table128 = _make_tc_repack(vocab, emb)(emb_table.T)
    gathered = _make_sc_gather(seq * batch, vocab, emb)(table128, idx_flat)
    emb_wide = gathered.reshape(seq, batch, _OUTW)

    bias = (b_ih + b_hh).reshape(hid, 1)
    out_t, emb_t = _make_tc_rnn(seq, batch, emb, hid)(
        emb_wide, W_ih, W_hh, bias
    )
    output_seq = out_t.transpose(0, 2, 1)
    embedded_seq = emb_t.transpose(0, 2, 1)
    last_hidden = output_seq[seq - 1 : seq]
    return output_seq, last_hidden, embedded_seq
